# baseline (speedup 1.0000x reference)
import functools
import math

import jax
import jax.numpy as jnp
import numpy as np
from jax import lax
from jax.experimental import pallas as pl
from jax.experimental.pallas import tpu as pltpu
from jax.experimental.pallas import tpu_sc as plsc

VOCAB = 100000
D_MODEL = 1024
BATCH = 4
SEQ = 2048

NC = 2
NS = 16
NW = NC * NS
LANES = 16

POS_PER_W = SEQ // NW
R = 16
H = POS_PER_W // R
NCHUNK = H * BATCH
NBUF = 4
DEPTH = 2
PEW = D_MODEL // 2
GROUPS_PER_ROW = D_MODEL // LANES


def _pe_packed() -> np.ndarray:
    pos = np.arange(SEQ, dtype=np.float64)[:, None]
    i = np.arange(0, D_MODEL, 2, dtype=np.float64)
    div = np.exp(-math.log(10000.0) * i / D_MODEL)
    pe = np.zeros((SEQ, D_MODEL), dtype=np.float64)
    pe[:, 0::2] = np.sin(pos * div)
    pe[:, 1::2] = np.cos(pos * div)
    return pe.astype(np.float32)


_PE_PACKED = _pe_packed()


@functools.partial(
    pl.kernel,
    out_type=jax.ShapeDtypeStruct((BATCH, SEQ, D_MODEL), jnp.float32),
    mesh=plsc.VectorSubcoreMesh(
        core_axis_name="c", subcore_axis_name="s", num_cores=NC,
        num_subcores=NS),
    scratch_types=[
        pltpu.VMEM((BATCH * POS_PER_W,), jnp.int32),
        [pltpu.VMEM((R, D_MODEL), jnp.float32)] * 2,
        [pltpu.VMEM((R, D_MODEL), jnp.float32)] * NBUF,
        [pltpu.SemaphoreType.DMA] * 2,
        [pltpu.SemaphoreType.DMA] * BATCH,
        [pltpu.SemaphoreType.DMA] * NBUF,
        [pltpu.SemaphoreType.DMA] * NBUF,
    ],
)
def _embed_sc(ids_hbm, table_hbm, pe_hbm, out_hbm,
              idx_v, pe_bufs, bufs, pe_sems, idsems, gsems, ssems):
    wid = lax.axis_index("s") * NC + lax.axis_index("c")
    pos_base = wid * POS_PER_W

    ids_copies = {}
    for b in range(BATCH):
        ids_copies[b] = pltpu.async_copy(
            ids_hbm.at[b, pl.ds(pos_base, POS_PER_W)],
            idx_v.at[pl.ds(b * POS_PER_W, POS_PER_W)],
            idsems[b],
        )

    def add_pe(buf, pe_v):
        def row_body(r, carry):
            for c in range(GROUPS_PER_ROW):
                plsc.addupdate(
                    buf.at[r, pl.ds(c * LANES, LANES)],
                    pe_v[r, pl.ds(c * LANES, LANES)],
                )
            return carry
        lax.fori_loop(0, R, row_body, 0)

    def issue_gather(k):
        h, b = divmod(k, BATCH)
        c = ids_copies.pop(b, None)
        if c is not None:
            c.wait()
        return pltpu.async_copy(
            table_hbm.at[idx_v.at[pl.ds(b * POS_PER_W + h * R, R)]],
            bufs[k % NBUF],
            gsems[k % NBUF],
        )

    def issue_pe(h):
        return pltpu.async_copy(
            pe_hbm.at[pl.ds(pos_base + h * R, R)],
            pe_bufs[h % 2],
            pe_sems[h % 2],
        )

    pe_copies = {0: issue_pe(0)}
    gathers = {k: issue_gather(k) for k in range(DEPTH)}
    stores = {}

    for k in range(NCHUNK):
        h, b = divmod(k, BATCH)
        if b == 0:
            pe_copies.pop(h).wait()
            if h + 1 < H:
                pe_copies[h + 1] = issue_pe(h + 1)
        if k + DEPTH < NCHUNK:
            prior = stores.pop(k - (NBUF - DEPTH), None)
            if prior is not None:
                prior.wait()
            gathers[k + DEPTH] = issue_gather(k + DEPTH)
        gathers.pop(k).wait()
        buf = bufs[k % NBUF]
        add_pe(buf, pe_bufs[h % 2])
        stores[k] = pltpu.async_copy(
            buf, out_hbm.at[b, pl.ds(pos_base + h * R, R)],
            ssems[k % NBUF],
        )
    for k in sorted(stores):
        stores.pop(k).wait()


def kernel(token_ids, token_table):
    ids = token_ids.astype(jnp.int32)
    pe = jnp.asarray(_PE_PACKED)
    return _embed_sc(ids, token_table, pe)


# speedup vs baseline: 1.0035x; 1.0035x over previous
import functools
import math

import jax
import jax.numpy as jnp
import numpy as np
from jax import lax
from jax.experimental import pallas as pl
from jax.experimental.pallas import tpu as pltpu
from jax.experimental.pallas import tpu_sc as plsc

VOCAB = 100000
D_MODEL = 1024
BATCH = 4
SEQ = 2048

NC = 2
NS = 16
NW = NC * NS
LANES = 16

POS_PER_W = SEQ // NW
R = 16
H = POS_PER_W // R
NCHUNK = H * BATCH
NBUF = 5
DEPTH = 3
PEW = D_MODEL // 2
GROUPS_PER_ROW = D_MODEL // LANES


def _pe_packed() -> np.ndarray:
    pos = np.arange(SEQ, dtype=np.float64)[:, None]
    i = np.arange(0, D_MODEL, 2, dtype=np.float64)
    div = np.exp(-math.log(10000.0) * i / D_MODEL)
    pe = np.zeros((SEQ, D_MODEL), dtype=np.float64)
    pe[:, 0::2] = np.sin(pos * div)
    pe[:, 1::2] = np.cos(pos * div)
    return pe.astype(np.float32)


_PE_PACKED = _pe_packed()


@functools.partial(
    pl.kernel,
    out_type=jax.ShapeDtypeStruct((BATCH, SEQ, D_MODEL), jnp.float32),
    mesh=plsc.VectorSubcoreMesh(
        core_axis_name="c", subcore_axis_name="s", num_cores=NC,
        num_subcores=NS),
    scratch_types=[
        pltpu.VMEM((BATCH * POS_PER_W,), jnp.int32),
        [pltpu.VMEM((R, D_MODEL), jnp.float32)] * 2,
        [pltpu.VMEM((R, D_MODEL), jnp.float32)] * NBUF,
        [pltpu.SemaphoreType.DMA] * 2,
        [pltpu.SemaphoreType.DMA] * BATCH,
        [pltpu.SemaphoreType.DMA] * NBUF,
        [pltpu.SemaphoreType.DMA] * NBUF,
    ],
)
def _embed_sc(ids_hbm, table_hbm, pe_hbm, out_hbm,
              idx_v, pe_bufs, bufs, pe_sems, idsems, gsems, ssems):
    wid = lax.axis_index("s") * NC + lax.axis_index("c")
    pos_base = wid * POS_PER_W

    ids_copies = {}
    for b in range(BATCH):
        ids_copies[b] = pltpu.async_copy(
            ids_hbm.at[b, pl.ds(pos_base, POS_PER_W)],
            idx_v.at[pl.ds(b * POS_PER_W, POS_PER_W)],
            idsems[b],
        )

    def add_pe(buf, pe_v):
        def row_body(r, carry):
            for c in range(GROUPS_PER_ROW):
                plsc.addupdate(
                    buf.at[r, pl.ds(c * LANES, LANES)],
                    pe_v[r, pl.ds(c * LANES, LANES)],
                )
            return carry
        lax.fori_loop(0, R, row_body, 0)

    def issue_gather(k):
        h, b = divmod(k, BATCH)
        c = ids_copies.pop(b, None)
        if c is not None:
            c.wait()
        return pltpu.async_copy(
            table_hbm.at[idx_v.at[pl.ds(b * POS_PER_W + h * R, R)]],
            bufs[k % NBUF],
            gsems[k % NBUF],
        )

    def issue_pe(h):
        return pltpu.async_copy(
            pe_hbm.at[pl.ds(pos_base + h * R, R)],
            pe_bufs[h % 2],
            pe_sems[h % 2],
        )

    pe_copies = {0: issue_pe(0)}
    gathers = {k: issue_gather(k) for k in range(DEPTH)}
    stores = {}

    for k in range(NCHUNK):
        h, b = divmod(k, BATCH)
        if b == 0:
            pe_copies.pop(h).wait()
            if h + 1 < H:
                pe_copies[h + 1] = issue_pe(h + 1)
        if k + DEPTH < NCHUNK:
            prior = stores.pop(k - (NBUF - DEPTH), None)
            if prior is not None:
                prior.wait()
            gathers[k + DEPTH] = issue_gather(k + DEPTH)
        gathers.pop(k).wait()
        buf = bufs[k % NBUF]
        add_pe(buf, pe_bufs[h % 2])
        stores[k] = pltpu.async_copy(
            buf, out_hbm.at[b, pl.ds(pos_base + h * R, R)],
            ssems[k % NBUF],
        )
    for k in sorted(stores):
        stores.pop(k).wait()


def kernel(token_ids, token_table):
    ids = token_ids.astype(jnp.int32)
    pe = jnp.asarray(_PE_PACKED)
    return _embed_sc(ids, token_table, pe)


# speedup vs baseline: 1.0048x; 1.0013x over previous
import functools
import math

import jax
import jax.numpy as jnp
import numpy as np
from jax import lax
from jax.experimental import pallas as pl
from jax.experimental.pallas import tpu as pltpu
from jax.experimental.pallas import tpu_sc as plsc

VOCAB = 100000
D_MODEL = 1024
BATCH = 4
SEQ = 2048

NC = 2
NS = 16
NW = NC * NS
LANES = 16

POS_PER_W = SEQ // NW
R = 16
H = POS_PER_W // R
NCHUNK = H * BATCH
NBUF = 5
DEPTH = 3
GROUPS_PER_ROW = D_MODEL // LANES


def _pe_const() -> np.ndarray:
    pos = np.arange(SEQ, dtype=np.float64)[:, None]
    i = np.arange(0, D_MODEL, 2, dtype=np.float64)
    div = np.exp(-math.log(10000.0) * i / D_MODEL)
    pe = np.zeros((SEQ, D_MODEL), dtype=np.float64)
    pe[:, 0::2] = np.sin(pos * div)
    pe[:, 1::2] = np.cos(pos * div)
    return pe.astype(np.float32)


_PE_CONST = _pe_const()


@functools.partial(
    pl.kernel,
    out_type=jax.ShapeDtypeStruct((BATCH, SEQ, D_MODEL), jnp.float32),
    mesh=plsc.VectorSubcoreMesh(
        core_axis_name="c", subcore_axis_name="s", num_cores=NC,
        num_subcores=NS),
    scratch_types=[
        pltpu.VMEM((BATCH * POS_PER_W,), jnp.int32),
        [pltpu.VMEM((R, D_MODEL), jnp.float32)] * 2,
        [pltpu.VMEM((R, D_MODEL), jnp.float32)] * NBUF,
        [pltpu.SemaphoreType.DMA] * 2,
        [pltpu.SemaphoreType.DMA] * BATCH,
        [pltpu.SemaphoreType.DMA] * NBUF,
        [pltpu.SemaphoreType.DMA] * NBUF,
    ],
)
def _embed_sc(ids_hbm, table_hbm, pe_hbm, out_hbm,
              idx_v, pe_bufs, bufs, pe_sems, idsems, gsems, ssems):
    wid = lax.axis_index("s") * NC + lax.axis_index("c")
    pos_base = wid * POS_PER_W

    ids_copies = {}
    for b in range(BATCH):
        ids_copies[b] = pltpu.async_copy(
            ids_hbm.at[b, pl.ds(pos_base, POS_PER_W)],
            idx_v.at[pl.ds(b * POS_PER_W, POS_PER_W)],
            idsems[b],
        )

    def add_pe(buf, pe_v):
        def row_body(r, carry):
            for c in range(GROUPS_PER_ROW):
                plsc.addupdate(
                    buf.at[r, pl.ds(c * LANES, LANES)],
                    pe_v[r, pl.ds(c * LANES, LANES)],
                )
            return carry
        lax.fori_loop(0, R, row_body, 0)

    def issue_gather(k):
        h, b = divmod(k, BATCH)
        c = ids_copies.pop(b, None)
        if c is not None:
            c.wait()
        return pltpu.async_copy(
            table_hbm.at[idx_v.at[pl.ds(b * POS_PER_W + h * R, R)]],
            bufs[k % NBUF],
            gsems[k % NBUF],
        )

    def issue_pe(h):
        return pltpu.async_copy(
            pe_hbm.at[pl.ds(pos_base + h * R, R)],
            pe_bufs[h % 2],
            pe_sems[h % 2],
        )

    pe_copies = {0: issue_pe(0)}
    gathers = {k: issue_gather(k) for k in range(DEPTH)}
    stores = {}

    for k in range(NCHUNK):
        h, b = divmod(k, BATCH)
        if b == 0:
            pe_copies.pop(h).wait()
            if h + 1 < H:
                pe_copies[h + 1] = issue_pe(h + 1)
        if k + DEPTH < NCHUNK:
            prior = stores.pop(k - (NBUF - DEPTH), None)
            if prior is not None:
                prior.wait()
            gathers[k + DEPTH] = issue_gather(k + DEPTH)
        gathers.pop(k).wait()
        buf = bufs[k % NBUF]
        add_pe(buf, pe_bufs[h % 2])
        stores[k] = pltpu.async_copy(
            buf, out_hbm.at[b, pl.ds(pos_base + h * R, R)],
            ssems[k % NBUF],
        )
    for k in sorted(stores):
        stores.pop(k).wait()


def kernel(token_ids, token_table):
    ids = token_ids.astype(jnp.int32)
    pe = jnp.asarray(_PE_CONST)
    return _embed_sc(ids, token_table, pe)
